# baseline (speedup 1.0000x reference)
import jax
import jax.numpy as jnp
from jax import lax
from jax.experimental import pallas as pl
from jax.experimental.pallas import tpu as pltpu
from jax.experimental.pallas import tpu_sc as plsc

_NC = 2
_NS = 16
_NW = _NC * _NS
_C = 128


def _gather_body(idx_hbm, table_hbm, out_hbm, idx_v, rows_v, sem):
    n_total = idx_hbm.shape[0]
    n_per_w = n_total // _NW
    n_chunks = n_per_w // _C
    wid = lax.axis_index("s") * _NC + lax.axis_index("c")
    base = wid * n_per_w

    @pl.loop(0, n_chunks)
    def _(j):
        off = base + j * _C
        pltpu.sync_copy(idx_hbm.at[pl.ds(off, _C)], idx_v)
        pltpu.async_copy(table_hbm.at[idx_v], rows_v, sem).wait()
        pltpu.sync_copy(rows_v, out_hbm.at[pl.ds(off, _C)])


def kernel(x, src_weight):
    batch, seq = x.shape
    _, dim = src_weight.shape
    n_total = batch * seq
    idx = x.reshape(n_total).astype(jnp.int32)

    mesh = plsc.VectorSubcoreMesh(core_axis_name="c", subcore_axis_name="s")
    out = pl.kernel(
        _gather_body,
        out_type=jax.ShapeDtypeStruct((n_total, dim), jnp.float32),
        mesh=mesh,
        scratch_types=[
            pltpu.VMEM((_C,), jnp.int32),
            pltpu.VMEM((_C, dim), jnp.float32),
            pltpu.SemaphoreType.DMA,
        ],
        compiler_params=pltpu.CompilerParams(use_tc_tiling_on_sc=False),
    )(idx, src_weight)
    return out.reshape(batch, seq, dim)


# speedup vs baseline: 1.1885x; 1.1885x over previous
import jax
import jax.numpy as jnp
from jax import lax
from jax.experimental import pallas as pl
from jax.experimental.pallas import tpu as pltpu
from jax.experimental.pallas import tpu_sc as plsc

_NC = 2
_NS = 16
_NW = _NC * _NS
_C = 128
_NB = 8


def _gather_body(idx_hbm, table_hbm, out_hbm, idx_v, rows_v, sem_i, sem_g, sem_o):
    n_total = idx_hbm.shape[0]
    n_per_w = n_total // _NW
    n_chunks = n_per_w // _C
    wid = lax.axis_index("s") * _NC + lax.axis_index("c")
    base = wid * n_per_w

    @pl.loop(0, n_chunks, step=_NB)
    def _(j0):
        loads = []
        for b in range(_NB):
            off = base + (j0 + b) * _C
            loads.append(pltpu.async_copy(
                idx_hbm.at[pl.ds(off, _C)], idx_v.at[b], sem_i.at[b]))
        gathers = []
        for b in range(_NB):
            loads[b].wait()
            gathers.append(pltpu.async_copy(
                table_hbm.at[idx_v.at[b]], rows_v.at[b], sem_g.at[b]))
        stores = []
        for b in range(_NB):
            off = base + (j0 + b) * _C
            gathers[b].wait()
            stores.append(pltpu.async_copy(
                rows_v.at[b], out_hbm.at[pl.ds(off, _C)], sem_o.at[b]))
        for b in range(_NB):
            stores[b].wait()


def kernel(x, src_weight):
    batch, seq = x.shape
    _, dim = src_weight.shape
    n_total = batch * seq
    idx = x.reshape(n_total).astype(jnp.int32)

    mesh = plsc.VectorSubcoreMesh(core_axis_name="c", subcore_axis_name="s")
    out = pl.kernel(
        _gather_body,
        out_type=jax.ShapeDtypeStruct((n_total, dim), jnp.float32),
        mesh=mesh,
        scratch_types=[
            pltpu.VMEM((_NB, _C), jnp.int32),
            pltpu.VMEM((_NB, _C, dim), jnp.float32),
            pltpu.SemaphoreType.DMA((_NB,)),
            pltpu.SemaphoreType.DMA((_NB,)),
            pltpu.SemaphoreType.DMA((_NB,)),
        ],
        compiler_params=pltpu.CompilerParams(use_tc_tiling_on_sc=False),
    )(idx, src_weight)
    return out.reshape(batch, seq, dim)


# speedup vs baseline: 1.1940x; 1.0046x over previous
import jax
import jax.numpy as jnp
from jax import lax
from jax.experimental import pallas as pl
from jax.experimental.pallas import tpu as pltpu
from jax.experimental.pallas import tpu_sc as plsc

_NC = 2
_NS = 16
_NW = _NC * _NS
_C = 128
_NB = 8


def _gather_body(idx_hbm, table_hbm, out_hbm, idx_v, rows_v, sem_i, sem_g, sem_o):
    n_total = idx_hbm.shape[0]
    n_per_w = n_total // _NW
    n_chunks = n_per_w // _C
    wid = lax.axis_index("s") * _NC + lax.axis_index("c")
    base = wid * n_per_w

    for b in range(_NB):
        pltpu.async_copy(
            idx_hbm.at[pl.ds(base + b * _C, _C)], idx_v.at[b], sem_i.at[b])

    @pl.loop(0, n_chunks, step=_NB)
    def _(j0):
        gathers = []
        for b in range(_NB):
            pltpu.make_async_copy(
                idx_hbm.at[pl.ds(base, _C)], idx_v.at[b], sem_i.at[b]).wait()
            gathers.append(pltpu.async_copy(
                table_hbm.at[idx_v.at[b]], rows_v.at[b], sem_g.at[b]))
        stores = []
        for b in range(_NB):
            gathers[b].wait()
            stores.append(pltpu.async_copy(
                rows_v.at[b], out_hbm.at[pl.ds(base + (j0 + b) * _C, _C)],
                sem_o.at[b]))
            off_next = lax.rem(j0 + _NB + b * 1, n_chunks) * _C + base
            pltpu.async_copy(
                idx_hbm.at[pl.ds(off_next, _C)], idx_v.at[b], sem_i.at[b])
        for b in range(_NB):
            stores[b].wait()

    for b in range(_NB):
        pltpu.make_async_copy(
            idx_hbm.at[pl.ds(base, _C)], idx_v.at[b], sem_i.at[b]).wait()


def kernel(x, src_weight):
    batch, seq = x.shape
    _, dim = src_weight.shape
    n_total = batch * seq
    idx = x.reshape(n_total).astype(jnp.int32)

    mesh = plsc.VectorSubcoreMesh(core_axis_name="c", subcore_axis_name="s")
    out = pl.kernel(
        _gather_body,
        out_type=jax.ShapeDtypeStruct((n_total, dim), jnp.float32),
        mesh=mesh,
        scratch_types=[
            pltpu.VMEM((_NB, _C), jnp.int32),
            pltpu.VMEM((_NB, _C, dim), jnp.float32),
            pltpu.SemaphoreType.DMA((_NB,)),
            pltpu.SemaphoreType.DMA((_NB,)),
            pltpu.SemaphoreType.DMA((_NB,)),
        ],
        compiler_params=pltpu.CompilerParams(use_tc_tiling_on_sc=False),
    )(idx, src_weight)
    return out.reshape(batch, seq, dim)
